# baseline (speedup 1.0000x reference)
import functools

import jax
import jax.numpy as jnp
from jax import lax
from jax.experimental import pallas as pl
from jax.experimental.pallas import tpu as pltpu
from jax.experimental.pallas import tpu_sc as plsc

_NC, _NS, _L = 2, 16, 16
_NW = _NC * _NS


def _segmax_sc(m2, src_e, dst_e):
    Nn, D2 = m2.shape
    E = src_e.shape[0]
    SEG = -(-Nn // (8 * _NW)) * 8
    NP = SEG * _NW
    C = 4000
    G = 96
    NCH = E // C
    NV = D2 // _L
    KS = NCH // _NW

    mesh = plsc.VectorSubcoreMesh(core_axis_name="c", subcore_axis_name="s")

    @functools.partial(
        pl.kernel,
        out_type=jax.ShapeDtypeStruct((NP, D2), jnp.float32),
        mesh=mesh,
        compiler_params=pltpu.CompilerParams(needs_layout_passes=False),
        scratch_types=[
            pltpu.VMEM((SEG + 8, D2), jnp.float32),
            pltpu.VMEM((C,), jnp.int32),
            pltpu.VMEM((C,), jnp.int32),
            pltpu.VMEM((C + G + _L,), jnp.int32),
            pltpu.VMEM((C + G + _L,), jnp.int32),
            pltpu.VMEM((G, D2), jnp.float32),
            pltpu.SemaphoreType.DMA,
        ],
    )
    def body(m_hbm, src_hbm, dst_hbm, out_hbm,
             acc, srcc, dstc, ssel, dsel, rows, sem):
        wid = lax.axis_index("s") * _NC + lax.axis_index("c")
        lo = wid * SEG
        zi = jnp.zeros((_L,), jnp.int32)
        zf = jnp.zeros((_L,), jnp.float32)
        lanes = lax.iota(jnp.int32, _L)

        def zrow(r, _):
            for t in range(NV):
                acc[r, pl.ds(t * _L, _L)] = zf
            return 0
        lax.fori_loop(0, SEG + 8, zrow, 0)

        def zsel(i, _):
            ssel[pl.ds(i * _L, _L)] = zi
            return 0
        lax.fori_loop(0, (C + G + _L) // _L, zsel, 0)

        def do_group(gb, dsafe):
            for l in range(_L):
                d = dsafe[l]
                avs = [acc[d, pl.ds(t * _L, _L)] for t in range(NV)]
                rvs = [rows[gb + l, pl.ds(t * _L, _L)] for t in range(NV)]
                for t in range(NV):
                    acc[d, pl.ds(t * _L, _L)] = jnp.maximum(avs[t], rvs[t])

        def chunk(ci, n):
            cj = (ci + wid * KS) % NCH
            e0 = cj * C
            pltpu.sync_copy(src_hbm.at[pl.ds(e0, C)], srcc)
            pltpu.sync_copy(dst_hbm.at[pl.ds(e0, C)], dstc)

            def filt(i, k):
                dv = dstc[pl.ds(i * _L, _L)]
                sv = srcc[pl.ds(i * _L, _L)]
                msk = (dv >= lo) & (dv < lo + SEG)
                mi = msk.astype(jnp.int32)
                cs = plsc.cumsum(mi)
                pos = (cs - mi) + k
                plsc.store_scatter(dsel, [pos], dv - lo, mask=msk)
                plsc.store_scatter(ssel, [pos], sv, mask=msk)
                return k + cs[_L - 1]
            k = lax.fori_loop(0, C // _L, filt, n)

            nb = k // G

            def gbatch(kb, _):
                pltpu.async_copy(
                    m_hbm.at[ssel.at[pl.ds(kb * G, G)]], rows, sem).wait()

                def group(g, _):
                    dvec = dsel[pl.ds(kb * G + g * _L, _L)]
                    do_group(g * _L, dvec)
                    return 0
                lax.fori_loop(0, G // _L, group, 0)
                return 0
            lax.fori_loop(0, nb, gbatch, 0)

            rem = k - nb * G

            def lcopy(i, _):
                ssel[pl.ds(i * _L, _L)] = ssel[pl.ds(nb * G + i * _L, _L)]
                dsel[pl.ds(i * _L, _L)] = dsel[pl.ds(nb * G + i * _L, _L)]
                return 0
            lax.fori_loop(0, (rem + _L - 1) // _L, lcopy, 0)
            return rem
        n = lax.fori_loop(0, NCH, chunk, 0)

        def fbatch(kb, _):
            pltpu.async_copy(
                m_hbm.at[ssel.at[pl.ds(kb * G, G)]], rows, sem).wait()

            def group(g, _):
                gb = kb * G + g * _L
                dvec = dsel[pl.ds(gb, _L)]
                dsafe = jnp.where((lanes + gb) < n, dvec, SEG)
                do_group(g * _L, dsafe)
                return 0
            lax.fori_loop(0, G // _L, group, 0)
            return 0
        lax.fori_loop(0, (n + G - 1) // G, fbatch, 0)

        pltpu.sync_copy(acc.at[pl.ds(0, SEG)], out_hbm.at[pl.ds(lo, SEG)])

    return body(m2, src_e, dst_e)


def _pool_body(x_ref, w_ref, b_ref, o_ref):
    D = w_ref.shape[1]
    o_ref[:, 0:D] = jax.nn.relu(
        jnp.dot(x_ref[0], w_ref[...], preferred_element_type=jnp.float32)
        + b_ref[...])
    o_ref[:, D:2 * D] = jax.nn.relu(
        jnp.dot(x_ref[1], w_ref[...], preferred_element_type=jnp.float32)
        + b_ref[...])


def _pool_features(x3, W, b, blk):
    Bn, Nn, D = x3.shape
    H = W.shape[1]
    grid = Nn // blk
    return pl.pallas_call(
        _pool_body,
        grid=(grid,),
        in_specs=[
            pl.BlockSpec((Bn, blk, D), lambda i: (0, i, 0)),
            pl.BlockSpec((D, H), lambda i: (0, 0)),
            pl.BlockSpec((1, H), lambda i: (0, 0)),
        ],
        out_specs=pl.BlockSpec((blk, Bn * H), lambda i: (i, 0)),
        out_shape=jax.ShapeDtypeStruct((Nn, Bn * H), jnp.float32),
    )(x3, W, b.reshape(1, H))


def _conv_body(x_ref, p_ref, ws_ref, wn_ref, b_ref, o_ref):
    acc = jnp.dot(x_ref[0], ws_ref[...], preferred_element_type=jnp.float32)
    acc += jnp.dot(p_ref[...], wn_ref[...], preferred_element_type=jnp.float32)
    o_ref[0] = jax.nn.relu(acc + b_ref[...])


def _conv_update(x3, pooled2, W_self, W_neigh, b_conv, blk):
    Bn, Nn, D = x3.shape
    H = W_self.shape[1]
    grid = (Bn, Nn // blk)
    return pl.pallas_call(
        _conv_body,
        grid=grid,
        in_specs=[
            pl.BlockSpec((1, blk, D), lambda b, i: (b, i, 0)),
            pl.BlockSpec((blk, D), lambda b, i: (i, b)),
            pl.BlockSpec((D, H), lambda b, i: (0, 0)),
            pl.BlockSpec((D, H), lambda b, i: (0, 0)),
            pl.BlockSpec((1, H), lambda b, i: (0, 0)),
        ],
        out_specs=pl.BlockSpec((1, blk, H), lambda b, i: (b, i, 0)),
        out_shape=jax.ShapeDtypeStruct((Bn, Nn, H), jnp.float32),
    )(x3, pooled2, W_self, W_neigh, b_conv.reshape(1, H))


def _heads_body(hb4_ref, hb2_ref, advw_ref, v1w_ref, advb_ref, v1b_ref,
                v2w_ref, v2b_ref, v3w_ref, v3b_ref, gmean_ref,
                o_ref, adv_acc, val_acc):
    i = pl.program_id(0)
    nsteps = pl.num_programs(0)
    A = adv_acc.shape[1]
    V = val_acc.shape[1]

    @pl.when(i == 0)
    def _init():
        adv_acc[...] = jnp.zeros_like(adv_acc)
        val_acc[...] = jnp.zeros_like(val_acc)

    aw = advw_ref[...]
    for j in range(4):
        t = jnp.dot(hb4_ref[j], aw, preferred_element_type=jnp.float32)
        adv_acc[...] += t[:, j * A:(j + 1) * A]
    vw = v1w_ref[...]
    for j in range(2):
        t = jnp.dot(hb2_ref[j], vw, preferred_element_type=jnp.float32)
        val_acc[...] += t[:, j * V:(j + 1) * V]

    @pl.when(i == nsteps - 1)
    def _fin():
        adv = jax.nn.relu(adv_acc[...] + advb_ref[...])
        v = jax.nn.relu(val_acc[...] + v1b_ref[...])
        v = jax.nn.relu(
            jnp.dot(v, v2w_ref[...], preferred_element_type=jnp.float32)
            + v2b_ref[...]
        )
        v = (jnp.dot(v, v3w_ref[...], preferred_element_type=jnp.float32)
             + v3b_ref[...])
        advm = jnp.dot(adv, gmean_ref[...], preferred_element_type=jnp.float32)
        o_ref[...] = v + adv - advm


def _heads(hb, adv_W, adv_b, val1_W, val1_b, val2_W, val2_b, val3_W, val3_b,
           n_groups, fblk):
    Bn, F = hb.shape
    A = adv_W.shape[1]
    V = val1_W.shape[1]
    grid = F // fblk
    f4, f2 = fblk // 4, fblk // 2
    ga = A // n_groups
    gidx = jnp.arange(A) // ga
    gmean = jnp.where(gidx[:, None] == gidx[None, :], 1.0 / ga, 0.0
                      ).astype(jnp.float32)
    adv4 = adv_W.reshape(F // 4, 4 * A)
    val2 = val1_W.reshape(F // 2, 2 * V)
    hb4 = hb.reshape(Bn, F // 4, 4).transpose(2, 0, 1)
    hb2 = hb.reshape(Bn, F // 2, 2).transpose(2, 0, 1)
    return pl.pallas_call(
        _heads_body,
        grid=(grid,),
        in_specs=[
            pl.BlockSpec((4, Bn, f4), lambda i: (0, 0, i)),
            pl.BlockSpec((2, Bn, f2), lambda i: (0, 0, i)),
            pl.BlockSpec((f4, 4 * A), lambda i: (i, 0)),
            pl.BlockSpec((f2, 2 * V), lambda i: (i, 0)),
            pl.BlockSpec((1, A), lambda i: (0, 0)),
            pl.BlockSpec((1, V), lambda i: (0, 0)),
            pl.BlockSpec((V, V), lambda i: (0, 0)),
            pl.BlockSpec((1, V), lambda i: (0, 0)),
            pl.BlockSpec((V, 1), lambda i: (0, 0)),
            pl.BlockSpec((1, 1), lambda i: (0, 0)),
            pl.BlockSpec((A, A), lambda i: (0, 0)),
        ],
        out_specs=pl.BlockSpec((Bn, A), lambda i: (0, 0)),
        out_shape=jax.ShapeDtypeStruct((Bn, A), jnp.float32),
        scratch_shapes=[
            pltpu.VMEM((Bn, A), jnp.float32),
            pltpu.VMEM((Bn, V), jnp.float32),
        ],
    )(hb4, hb2, adv4, val2, adv_b.reshape(1, A), val1_b.reshape(1, V),
      val2_W, val2_b.reshape(1, V), val3_W, val3_b.reshape(1, 1), gmean)


def kernel(x, edge_index, W_pool, b_pool, W_self, W_neigh, b_conv,
           adv_W, adv_b, val1_W, val1_b, val2_W, val2_b, val3_W, val3_b):
    Bn, Nn, Dd = x.shape
    Hh = W_self.shape[1]
    NA_groups = 4

    m2 = _pool_features(x, W_pool, b_pool, blk=1000)

    pooled2 = _segmax_sc(m2, edge_index[0], edge_index[1])

    h3 = _conv_update(x, pooled2, W_self, W_neigh, b_conv, blk=1000)
    hb = h3.reshape(Bn, Nn * Hh)

    q32 = _heads(hb, adv_W, adv_b, val1_W, val1_b, val2_W, val2_b,
                 val3_W, val3_b, n_groups=NA_groups, fblk=25600)
    A = adv_W.shape[1]
    return q32.reshape(Bn, NA_groups, A // NA_groups)


# speedup vs baseline: 1.4910x; 1.4910x over previous
import functools

import jax
import jax.numpy as jnp
from jax import lax
from jax.experimental import pallas as pl
from jax.experimental.pallas import tpu as pltpu
from jax.experimental.pallas import tpu_sc as plsc

_NC, _NS, _L = 2, 16, 16
_NW = _NC * _NS


def _segmax_sc(m2, src_e, dst_e):
    Nn, D2 = m2.shape
    E = src_e.shape[0]
    SEG = -(-Nn // (8 * _NW)) * 8
    NP = SEG * _NW
    C = 4000
    G = 96
    NCH = E // C
    NV = D2 // _L
    KS = NCH // _NW

    mesh = plsc.VectorSubcoreMesh(core_axis_name="c", subcore_axis_name="s")

    @functools.partial(
        pl.kernel,
        out_type=jax.ShapeDtypeStruct((NP, D2), jnp.float32),
        mesh=mesh,
        compiler_params=pltpu.CompilerParams(needs_layout_passes=False),
        scratch_types=[
            pltpu.VMEM((SEG + 8, D2), jnp.float32),
            pltpu.VMEM((C,), jnp.int32),
            pltpu.VMEM((C,), jnp.int32),
            pltpu.VMEM((C + G + _L,), jnp.int32),
            pltpu.VMEM((C + G + _L,), jnp.int32),
            pltpu.VMEM((G, D2), jnp.float32),
            pltpu.SemaphoreType.DMA,
        ],
    )
    def body(m_hbm, src_hbm, dst_hbm, out_hbm,
             acc, srcc, dstc, ssel, dsel, rows, sem):
        wid = lax.axis_index("s") * _NC + lax.axis_index("c")
        lo = wid * SEG
        zi = jnp.zeros((_L,), jnp.int32)
        zf = jnp.zeros((_L,), jnp.float32)
        lanes = lax.iota(jnp.int32, _L)

        def zrow(r, _):
            for t in range(NV):
                acc[r, pl.ds(t * _L, _L)] = zf
            return 0
        lax.fori_loop(0, SEG + 8, zrow, 0)

        def zsel(i, _):
            ssel[pl.ds(i * _L, _L)] = zi
            return 0
        lax.fori_loop(0, (C + G + _L) // _L, zsel, 0)

        def do_group(gb, dsafe):
            for l in range(_L):
                d = dsafe[l]
                avs = [acc[d, pl.ds(t * _L, _L)] for t in range(NV)]
                rvs = [rows[gb + l, pl.ds(t * _L, _L)] for t in range(NV)]
                for t in range(NV):
                    acc[d, pl.ds(t * _L, _L)] = jnp.maximum(avs[t], rvs[t])

        def chunk(ci, n):
            cj = (ci + wid * KS) % NCH
            e0 = cj * C
            pltpu.sync_copy(src_hbm.at[pl.ds(e0, C)], srcc)
            pltpu.sync_copy(dst_hbm.at[pl.ds(e0, C)], dstc)

            def filt(i, k):
                dv = dstc[pl.ds(i * _L, _L)]
                sv = srcc[pl.ds(i * _L, _L)]
                msk = (dv >= lo) & (dv < lo + SEG)
                mi = msk.astype(jnp.int32)
                cs = plsc.cumsum(mi)
                pos = (cs - mi) + k
                plsc.store_scatter(dsel, [pos], dv - lo, mask=msk)
                plsc.store_scatter(ssel, [pos], sv, mask=msk)
                return k + cs[_L - 1]
            k = lax.fori_loop(0, C // _L, filt, n)

            nb = k // G

            def gbatch(kb, _):
                pltpu.async_copy(
                    m_hbm.at[ssel.at[pl.ds(kb * G, G)]], rows, sem).wait()

                def group(g, _):
                    dvec = dsel[pl.ds(kb * G + g * _L, _L)]
                    do_group(g * _L, dvec)
                    return 0
                lax.fori_loop(0, G // _L, group, 0)
                return 0
            lax.fori_loop(0, nb, gbatch, 0)

            rem = k - nb * G

            def lcopy(i, _):
                ssel[pl.ds(i * _L, _L)] = ssel[pl.ds(nb * G + i * _L, _L)]
                dsel[pl.ds(i * _L, _L)] = dsel[pl.ds(nb * G + i * _L, _L)]
                return 0
            lax.fori_loop(0, (rem + _L - 1) // _L, lcopy, 0)
            return rem
        n = lax.fori_loop(0, NCH, chunk, 0)

        def fbatch(kb, _):
            pltpu.async_copy(
                m_hbm.at[ssel.at[pl.ds(kb * G, G)]], rows, sem).wait()

            def group(g, _):
                gb = kb * G + g * _L
                dvec = dsel[pl.ds(gb, _L)]
                dsafe = jnp.where((lanes + gb) < n, dvec, SEG)
                do_group(g * _L, dsafe)
                return 0
            lax.fori_loop(0, G // _L, group, 0)
            return 0
        lax.fori_loop(0, (n + G - 1) // G, fbatch, 0)

        pltpu.sync_copy(acc.at[pl.ds(0, SEG)], out_hbm.at[pl.ds(lo, SEG)])

    return body(m2, src_e, dst_e)


def _pool_body(x_ref, w_ref, b_ref, o_ref):
    D = w_ref.shape[1]
    o_ref[:, 0:D] = jax.nn.relu(
        jnp.dot(x_ref[0], w_ref[...], preferred_element_type=jnp.float32)
        + b_ref[...])
    o_ref[:, D:2 * D] = jax.nn.relu(
        jnp.dot(x_ref[1], w_ref[...], preferred_element_type=jnp.float32)
        + b_ref[...])


def _pool_features(x3, W, b, blk):
    Bn, Nn, D = x3.shape
    H = W.shape[1]
    grid = Nn // blk
    return pl.pallas_call(
        _pool_body,
        grid=(grid,),
        in_specs=[
            pl.BlockSpec((Bn, blk, D), lambda i: (0, i, 0)),
            pl.BlockSpec((D, H), lambda i: (0, 0)),
            pl.BlockSpec((1, H), lambda i: (0, 0)),
        ],
        out_specs=pl.BlockSpec((blk, Bn * H), lambda i: (i, 0)),
        out_shape=jax.ShapeDtypeStruct((Nn, Bn * H), jnp.float32),
    )(x3, W, b.reshape(1, H))


def _conv_body(x_ref, p_ref, ws_ref, wn_ref, b_ref, o_ref):
    acc = jnp.dot(x_ref[0], ws_ref[...], preferred_element_type=jnp.float32)
    acc += jnp.dot(p_ref[...], wn_ref[...], preferred_element_type=jnp.float32)
    o_ref[0] = jax.nn.relu(acc + b_ref[...])


def _conv_update(x3, pooled2, W_self, W_neigh, b_conv, blk):
    Bn, Nn, D = x3.shape
    H = W_self.shape[1]
    grid = (Bn, Nn // blk)
    return pl.pallas_call(
        _conv_body,
        grid=grid,
        in_specs=[
            pl.BlockSpec((1, blk, D), lambda b, i: (b, i, 0)),
            pl.BlockSpec((blk, D), lambda b, i: (i, b)),
            pl.BlockSpec((D, H), lambda b, i: (0, 0)),
            pl.BlockSpec((D, H), lambda b, i: (0, 0)),
            pl.BlockSpec((1, H), lambda b, i: (0, 0)),
        ],
        out_specs=pl.BlockSpec((1, blk, H), lambda b, i: (b, i, 0)),
        out_shape=jax.ShapeDtypeStruct((Bn, Nn, H), jnp.float32),
    )(x3, pooled2, W_self, W_neigh, b_conv.reshape(1, H))


def _heads_body(hb_ref, advw_ref, v1w_ref, advb_ref, v1b_ref,
                v2w_ref, v2b_ref, v3w_ref, v3b_ref, gmean_ref,
                o_ref, adv_acc, val_acc):
    i = pl.program_id(0)
    nsteps = pl.num_programs(0)

    @pl.when(i == 0)
    def _init():
        adv_acc[...] = jnp.zeros_like(adv_acc)
        val_acc[...] = jnp.zeros_like(val_acc)

    hb = hb_ref[...]
    adv_acc[...] += jnp.dot(hb, advw_ref[...], preferred_element_type=jnp.float32)
    val_acc[...] += jnp.dot(hb, v1w_ref[...], preferred_element_type=jnp.float32)

    @pl.when(i == nsteps - 1)
    def _fin():
        adv = jax.nn.relu(adv_acc[...] + advb_ref[...])
        v = jax.nn.relu(val_acc[...] + v1b_ref[...])
        v = jax.nn.relu(
            jnp.dot(v, v2w_ref[...], preferred_element_type=jnp.float32)
            + v2b_ref[...]
        )
        v = (jnp.dot(v, v3w_ref[...], preferred_element_type=jnp.float32)
             + v3b_ref[...])
        advm = jnp.dot(adv, gmean_ref[...], preferred_element_type=jnp.float32)
        o_ref[...] = v + adv - advm


def _heads(hb, adv_W, adv_b, val1_W, val1_b, val2_W, val2_b, val3_W, val3_b,
           n_groups, fblk):
    Bn, F = hb.shape
    A = adv_W.shape[1]
    V = val1_W.shape[1]
    grid = F // fblk
    ga = A // n_groups
    gidx = jnp.arange(A) // ga
    gmean = jnp.where(gidx[:, None] == gidx[None, :], 1.0 / ga, 0.0
                      ).astype(jnp.float32)
    return pl.pallas_call(
        _heads_body,
        grid=(grid,),
        in_specs=[
            pl.BlockSpec((Bn, fblk), lambda i: (0, i)),
            pl.BlockSpec((fblk, A), lambda i: (i, 0)),
            pl.BlockSpec((fblk, V), lambda i: (i, 0)),
            pl.BlockSpec((1, A), lambda i: (0, 0)),
            pl.BlockSpec((1, V), lambda i: (0, 0)),
            pl.BlockSpec((V, V), lambda i: (0, 0)),
            pl.BlockSpec((1, V), lambda i: (0, 0)),
            pl.BlockSpec((V, 1), lambda i: (0, 0)),
            pl.BlockSpec((1, 1), lambda i: (0, 0)),
            pl.BlockSpec((A, A), lambda i: (0, 0)),
        ],
        out_specs=pl.BlockSpec((Bn, A), lambda i: (0, 0)),
        out_shape=jax.ShapeDtypeStruct((Bn, A), jnp.float32),
        scratch_shapes=[
            pltpu.VMEM((Bn, A), jnp.float32),
            pltpu.VMEM((Bn, V), jnp.float32),
        ],
    )(hb, adv_W, val1_W, adv_b.reshape(1, A), val1_b.reshape(1, V),
      val2_W, val2_b.reshape(1, V), val3_W, val3_b.reshape(1, 1), gmean)


def kernel(x, edge_index, W_pool, b_pool, W_self, W_neigh, b_conv,
           adv_W, adv_b, val1_W, val1_b, val2_W, val2_b, val3_W, val3_b):
    Bn, Nn, Dd = x.shape
    Hh = W_self.shape[1]
    NA_groups = 4

    m2 = _pool_features(x, W_pool, b_pool, blk=1000)

    pooled2 = _segmax_sc(m2, edge_index[0], edge_index[1])

    h3 = _conv_update(x, pooled2, W_self, W_neigh, b_conv, blk=1000)
    hb = h3.reshape(Bn, Nn * Hh)

    q32 = _heads(hb, adv_W, adv_b, val1_W, val1_b, val2_W, val2_b,
                 val3_W, val3_b, n_groups=NA_groups, fblk=16000)
    A = adv_W.shape[1]
    return q32.reshape(Bn, NA_groups, A // NA_groups)
